# baseline (speedup 1.0000x reference)
import jax
import jax.numpy as jnp
from jax import lax
from jax.experimental import pallas as pl
from jax.experimental.pallas import tpu as pltpu
from jax.experimental.pallas import tpu_sc as plsc

BB, SS, DD = 4, 8192, 768
EE, DFF, OUT = 64, 64, 768
TT = BB * SS
CAP = TT // EE
NOISE_STD = 1.0 / EE
RA = 2048
NA = TT // RA
RB = 512
NB = TT // RB
NCLS = 8
PAD = 2048
PREFIX = EE * CAP + PAD
NC = PREFIX + TT
NDISP = EE * CAP + 8
CHUNK = 64
SQRT_HALF = 0.7071067811865476


def _stage_a_body(x_ref, noise_ref, gw_ref, gb_ref, ls_ref, lb_ref,
                  comb_ref, aux_ref, imp_acc, p_acc):
    i = pl.program_id(0)
    xb = x_ref[...]
    mu = jnp.mean(xb, axis=1, keepdims=True)
    xc = xb - mu
    var = jnp.mean(xc * xc, axis=1, keepdims=True)
    xn = xc / jnp.sqrt(var + 1e-5) * ls_ref[...] + lb_ref[...]
    comb_ref[...] = xn
    logits = lax.dot_general(
        xn.astype(jnp.bfloat16), gw_ref[...].astype(jnp.bfloat16),
        (((1,), (0,)), ((), ())), preferred_element_type=jnp.float32)
    logits = logits + gb_ref[...]

    @pl.when(i == 0)
    def _init():
        imp_acc[...] = jnp.zeros_like(imp_acc)
        p_acc[...] = jnp.zeros_like(p_acc)

    m = jnp.max(logits, axis=1, keepdims=True)
    ex = jnp.exp(logits - m)
    gates = ex / jnp.sum(ex, axis=1, keepdims=True)
    imp_acc[...] += jnp.sum(gates, axis=0, keepdims=True)

    noisy = logits + noise_ref[...]
    thr = jnp.max(noisy, axis=1, keepdims=True)
    z = (thr - logits) * (1.0 / NOISE_STD)
    p = 0.5 * (1.0 - lax.erf(z * SQRT_HALF))
    p_acc[...] += jnp.sum(p, axis=0, keepdims=True)

    @pl.when(i == NA - 1)
    def _fin():
        def cv2(v):
            mean = jnp.mean(v)
            ss = jnp.sum((v - mean) ** 2) / (EE - 1)
            return ss / (mean + 1e-8) ** 2
        imp_loss = cv2(imp_acc[...])
        load_loss = cv2(p_acc[...] / TT)
        aux_ref[...] = jnp.full((1, 1), 0.5, jnp.float32) * (imp_loss + load_loss)


def _run_stage_a(xf, noise, gate_W, gate_b, ln_scale, ln_bias):
    return pl.pallas_call(
        _stage_a_body,
        grid=(NA,),
        in_specs=[
            pl.BlockSpec((RA, DD), lambda i: (i, 0)),
            pl.BlockSpec((RA, EE), lambda i: (i, 0)),
            pl.BlockSpec((DD, EE), lambda i: (0, 0)),
            pl.BlockSpec((1, EE), lambda i: (0, 0)),
            pl.BlockSpec((1, DD), lambda i: (0, 0)),
            pl.BlockSpec((1, DD), lambda i: (0, 0)),
        ],
        out_specs=[
            pl.BlockSpec((RA, DD), lambda i: (i + PREFIX // RA, 0)),
            pl.BlockSpec((1, 1), lambda i: (0, 0)),
        ],
        out_shape=[
            jax.ShapeDtypeStruct((NC, DD), jnp.float32),
            jax.ShapeDtypeStruct((1, 1), jnp.float32),
        ],
        scratch_shapes=[
            pltpu.VMEM((1, EE), jnp.float32),
            pltpu.VMEM((1, EE), jnp.float32),
        ],
    )(xf, noise, gate_W, gate_b.reshape(1, EE),
      ln_scale.reshape(1, DD), ln_bias.reshape(1, DD))


def _cls_of(w):
    one = jnp.int32(0x3F800000)
    k = one - lax.bitcast_convert_type(w, jnp.int32)
    return jnp.clip(k + 1, 0, NCLS - 1)


def _stage_b1_body(eid_ref, w_ref, pfx_ref, hist):
    i = pl.program_id(0)

    @pl.when(i == 0)
    def _init():
        hist[...] = jnp.zeros_like(hist)

    pfx_ref[...] = hist[...].reshape(1, EE, NCLS)

    @pl.when(i < NB)
    def _accum():
        eid = eid_ref[...].reshape(1, RB)
        cls = _cls_of(w_ref[...].reshape(1, RB))
        iota_e = lax.broadcasted_iota(jnp.int32, (EE, 1), 0)
        oh = (eid == iota_e).astype(jnp.float32)
        for c in range(NCLS):
            mc = (cls == c).astype(jnp.float32)
            hist[:, c:c + 1] += jnp.sum(oh * mc, axis=1, keepdims=True)


def _run_stage_b1(eid, w):
    return pl.pallas_call(
        _stage_b1_body,
        grid=(NB + 1,),
        in_specs=[
            pl.BlockSpec((1, 1, RB), lambda i: (jnp.minimum(i, NB - 1), 0, 0)),
            pl.BlockSpec((1, 1, RB), lambda i: (jnp.minimum(i, NB - 1), 0, 0)),
        ],
        out_specs=pl.BlockSpec((1, EE, NCLS), lambda i: (i, 0, 0)),
        out_shape=jax.ShapeDtypeStruct((NB + 1, EE, NCLS), jnp.float32),
        scratch_shapes=[pltpu.VMEM((EE, NCLS), jnp.float32)],
    )(eid, w)


def _stage_b2_body(eid_ref, w_ref, pfx_ref, tot_ref, gidx_ref):
    eid = eid_ref[...].reshape(1, RB)
    cls = _cls_of(w_ref[...].reshape(1, RB))
    iota_e = lax.broadcasted_iota(jnp.int32, (EE, 1), 0)
    oh = (eid == iota_e).astype(jnp.float32)
    ia = lax.broadcasted_iota(jnp.int32, (RB, RB), 0)
    ib = lax.broadcasted_iota(jnp.int32, (RB, RB), 1)
    utri = (ia < ib).astype(jnp.float32)

    pfx = pfx_ref[0]
    tot = tot_ref[0]
    base = jnp.zeros((1, RB), jnp.float32)
    within = jnp.zeros((1, RB), jnp.float32)
    higher = jnp.zeros((EE, 1), jnp.float32)
    for c in range(NCLS):
        row = higher + pfx[:, c:c + 1]
        mc = (cls == c).astype(jnp.float32)
        base += mc * jnp.sum(oh * row, axis=0, keepdims=True)
        ohc = oh * mc
        cumc = lax.dot_general(ohc, utri, (((1,), (0,)), ((), ())),
                               preferred_element_type=jnp.float32)
        within += jnp.sum(ohc * cumc, axis=0, keepdims=True)
        higher = higher + tot[:, c:c + 1]
    slot = base + within
    keep = slot < float(CAP)
    gidx = jnp.where(keep, eid.astype(jnp.float32) * CAP + slot,
                     float(EE * CAP)).astype(jnp.int32)
    gidx_ref[...] = gidx.reshape(1, 1, RB)


def _run_stage_b2(eid, w, pfx):
    return pl.pallas_call(
        _stage_b2_body,
        grid=(NB,),
        in_specs=[
            pl.BlockSpec((1, 1, RB), lambda i: (i, 0, 0)),
            pl.BlockSpec((1, 1, RB), lambda i: (i, 0, 0)),
            pl.BlockSpec((1, EE, NCLS), lambda i: (i, 0, 0)),
            pl.BlockSpec((1, EE, NCLS), lambda i: (NB, 0, 0)),
        ],
        out_specs=pl.BlockSpec((1, 1, RB), lambda i: (i, 0, 0)),
        out_shape=jax.ShapeDtypeStruct((NB, 1, RB), jnp.int32),
    )(eid, w, pfx, pfx)


def _sc_dispatch_body(comb_hbm, gidx_hbm, disp_hbm, idx_v, rows_v, sem):
    wid = lax.axis_index("s") * 2 + lax.axis_index("c")
    rows_per_tile = TT // 32

    def body(j, carry):
        base = pl.multiple_of(wid * rows_per_tile + j * CHUNK, CHUNK)
        pltpu.sync_copy(gidx_hbm.at[pl.ds(base, CHUNK)], idx_v)
        pltpu.sync_copy(comb_hbm.at[pl.ds(PREFIX + base, CHUNK)], rows_v)
        pltpu.async_copy(rows_v, disp_hbm.at[idx_v], sem).wait()
        return carry

    lax.fori_loop(0, rows_per_tile // CHUNK, body, 0)


def _run_stage_c(comb, gidx_flat):
    mesh = plsc.VectorSubcoreMesh(core_axis_name="c", subcore_axis_name="s")
    f = pl.kernel(
        _sc_dispatch_body,
        out_type=jax.ShapeDtypeStruct((NDISP, DD), jnp.float32),
        mesh=mesh,
        scratch_types=[
            pltpu.VMEM((CHUNK,), jnp.int32),
            pltpu.VMEM((CHUNK, DD), jnp.float32),
            pltpu.SemaphoreType.DMA,
        ],
    )
    return f(comb, gidx_flat)


def _stage_d_body(comb_in_ref, disp_ref, w1_ref, b1_ref, w2_ref, b2_ref,
                  comb_out_ref):
    del comb_in_ref
    xb = disp_ref[...]
    u = lax.dot_general(
        xb.astype(jnp.bfloat16), w1_ref[0].astype(jnp.bfloat16),
        (((1,), (0,)), ((), ())), preferred_element_type=jnp.float32)
    u = u + b1_ref[0]
    h = jax.nn.gelu(u)
    o = lax.dot_general(
        h.astype(jnp.bfloat16), w2_ref[0].astype(jnp.bfloat16),
        (((1,), (0,)), ((), ())), preferred_element_type=jnp.float32)
    comb_out_ref[...] = o + b2_ref[0]


def _run_stage_d(comb, disp, W1, b1, W2, b2):
    return pl.pallas_call(
        _stage_d_body,
        grid=(EE,),
        in_specs=[
            pl.BlockSpec(memory_space=pl.ANY),
            pl.BlockSpec((CAP, DD), lambda e: (e, 0)),
            pl.BlockSpec((1, DD, DFF), lambda e: (e, 0, 0)),
            pl.BlockSpec((1, 1, DFF), lambda e: (e, 0, 0)),
            pl.BlockSpec((1, DFF, OUT), lambda e: (e, 0, 0)),
            pl.BlockSpec((1, 1, OUT), lambda e: (e, 0, 0)),
        ],
        out_specs=pl.BlockSpec((CAP, OUT), lambda e: (e, 0)),
        out_shape=jax.ShapeDtypeStruct((NC, OUT), jnp.float32),
        input_output_aliases={0: 0},
    )(comb, disp, W1, b1.reshape(EE, 1, DFF), W2, b2.reshape(EE, 1, OUT))


def _sc_combine_body(comb_hbm, gidx_hbm, out_hbm, idx_v, g3_v, rows_v, sem):
    wid = lax.axis_index("s") * 2 + lax.axis_index("c")
    rows_per_tile = TT // 32

    def body(j, carry):
        base = pl.multiple_of(wid * rows_per_tile + j * CHUNK, CHUNK)
        pltpu.sync_copy(gidx_hbm.at[pl.ds(base, CHUNK)], idx_v)
        for i in range(CHUNK // 16):
            v = idx_v[pl.ds(i * 16, 16)]
            tok = base + i * 16 + lax.iota(jnp.int32, 16)
            g3_v[pl.ds(i * 16, 16)] = jnp.where(v < EE * CAP, v, PREFIX + tok)
        pltpu.async_copy(comb_hbm.at[g3_v], rows_v, sem).wait()
        pltpu.sync_copy(rows_v, out_hbm.at[pl.ds(base, CHUNK)])
        return carry

    lax.fori_loop(0, rows_per_tile // CHUNK, body, 0)


def _run_stage_e(comb2, gidx_flat):
    mesh = plsc.VectorSubcoreMesh(core_axis_name="c", subcore_axis_name="s")
    f = pl.kernel(
        _sc_combine_body,
        out_type=jax.ShapeDtypeStruct((TT, OUT), jnp.float32),
        mesh=mesh,
        scratch_types=[
            pltpu.VMEM((CHUNK,), jnp.int32),
            pltpu.VMEM((CHUNK,), jnp.int32),
            pltpu.VMEM((CHUNK, OUT), jnp.float32),
            pltpu.SemaphoreType.DMA,
        ],
    )
    return f(comb2, gidx_flat)


def kernel(x, gate_W, gate_b, ln_scale, ln_bias, W1, b1, W2, b2):
    xf = x.reshape(TT, DD)
    noise = NOISE_STD * jax.random.normal(
        jax.random.key(42), (TT, EE), dtype=jnp.float32)

    comb, aux = _run_stage_a(xf, noise, gate_W, gate_b, ln_scale, ln_bias)

    mu = xf.mean(-1, keepdims=True)
    var = ((xf - mu) ** 2).mean(-1, keepdims=True)
    xn_r = (xf - mu) / jnp.sqrt(var + 1e-5) * ln_scale + ln_bias
    logits = xn_r @ gate_W + gate_b
    gates_noisy = jax.nn.softmax(logits + noise, axis=-1)
    topv, topi = jax.lax.top_k(gates_noisy, 1)
    wsel = (topv / (topv.sum(-1, keepdims=True) + 1e-20)).reshape(NB, 1, RB)
    eid = topi.astype(jnp.int32).reshape(NB, 1, RB)

    pfx = _run_stage_b1(eid, wsel)
    gidx = _run_stage_b2(eid, wsel, pfx)
    gidx_flat = gidx.reshape(TT)
    disp = _run_stage_c(comb, gidx_flat)
    comb2 = _run_stage_d(comb, disp, W1, b1, W2, b2)
    out = _run_stage_e(comb2, gidx_flat)
    return out.reshape(BB, SS, OUT), aux[0, 0]


# speedup vs baseline: 6.6828x; 6.6828x over previous
import jax
import jax.numpy as jnp
from jax import lax
from jax.experimental import pallas as pl
from jax.experimental.pallas import tpu as pltpu
from jax.experimental.pallas import tpu_sc as plsc

BB, SS, DD = 4, 8192, 768
EE, DFF, OUT = 64, 64, 768
TT = BB * SS
CAP = TT // EE
NOISE_STD = 1.0 / EE
RA = 2048
NA = TT // RA
RB = 512
NB = TT // RB
NCLS = 8
PAD = 2048
PREFIX = EE * CAP + PAD
NC = PREFIX + TT
NDISP = EE * CAP + 8
CHUNK = 64
SQRT_HALF = 0.7071067811865476


def _stage_a_body(x_ref, noise_ref, gw_ref, gb_ref, ls_ref, lb_ref,
                  comb_ref, aux_ref, imp_acc, p_acc):
    i = pl.program_id(0)
    xb = x_ref[...]
    mu = jnp.mean(xb, axis=1, keepdims=True)
    xc = xb - mu
    var = jnp.mean(xc * xc, axis=1, keepdims=True)
    xn = xc / jnp.sqrt(var + 1e-5) * ls_ref[...] + lb_ref[...]
    comb_ref[...] = xn
    logits = lax.dot_general(
        xn.astype(jnp.bfloat16), gw_ref[...].astype(jnp.bfloat16),
        (((1,), (0,)), ((), ())), preferred_element_type=jnp.float32)
    logits = logits + gb_ref[...]

    @pl.when(i == 0)
    def _init():
        imp_acc[...] = jnp.zeros_like(imp_acc)
        p_acc[...] = jnp.zeros_like(p_acc)

    m = jnp.max(logits, axis=1, keepdims=True)
    ex = jnp.exp(logits - m)
    gates = ex / jnp.sum(ex, axis=1, keepdims=True)
    imp_acc[...] += jnp.sum(gates, axis=0, keepdims=True)

    noisy = logits + noise_ref[...]
    thr = jnp.max(noisy, axis=1, keepdims=True)
    z = (thr - logits) * (1.0 / NOISE_STD)
    p = 0.5 * (1.0 - lax.erf(z * SQRT_HALF))
    p_acc[...] += jnp.sum(p, axis=0, keepdims=True)

    @pl.when(i == NA - 1)
    def _fin():
        def cv2(v):
            mean = jnp.mean(v)
            ss = jnp.sum((v - mean) ** 2) / (EE - 1)
            return ss / (mean + 1e-8) ** 2
        imp_loss = cv2(imp_acc[...])
        load_loss = cv2(p_acc[...] / TT)
        aux_ref[...] = jnp.full((1, 1), 0.5, jnp.float32) * (imp_loss + load_loss)


def _run_stage_a(xf, noise, gate_W, gate_b, ln_scale, ln_bias):
    return pl.pallas_call(
        _stage_a_body,
        grid=(NA,),
        in_specs=[
            pl.BlockSpec((RA, DD), lambda i: (i, 0)),
            pl.BlockSpec((RA, EE), lambda i: (i, 0)),
            pl.BlockSpec((DD, EE), lambda i: (0, 0)),
            pl.BlockSpec((1, EE), lambda i: (0, 0)),
            pl.BlockSpec((1, DD), lambda i: (0, 0)),
            pl.BlockSpec((1, DD), lambda i: (0, 0)),
        ],
        out_specs=[
            pl.BlockSpec((RA, DD), lambda i: (i + PREFIX // RA, 0)),
            pl.BlockSpec((1, 1), lambda i: (0, 0)),
        ],
        out_shape=[
            jax.ShapeDtypeStruct((NC, DD), jnp.float32),
            jax.ShapeDtypeStruct((1, 1), jnp.float32),
        ],
        scratch_shapes=[
            pltpu.VMEM((1, EE), jnp.float32),
            pltpu.VMEM((1, EE), jnp.float32),
        ],
    )(xf, noise, gate_W, gate_b.reshape(1, EE),
      ln_scale.reshape(1, DD), ln_bias.reshape(1, DD))


def _cls_of(w):
    one = jnp.int32(0x3F800000)
    k = one - lax.bitcast_convert_type(w, jnp.int32)
    return jnp.clip(k + 1, 0, NCLS - 1)


def _stage_b1_body(eid_ref, w_ref, pfx_ref, hist):
    i = pl.program_id(0)

    @pl.when(i == 0)
    def _init():
        hist[...] = jnp.zeros_like(hist)

    pfx_ref[...] = hist[...].reshape(1, EE, NCLS)

    @pl.when(i < NB)
    def _accum():
        eid = eid_ref[...].reshape(1, RB)
        cls = _cls_of(w_ref[...].reshape(1, RB))
        iota_e = lax.broadcasted_iota(jnp.int32, (EE, 1), 0)
        oh = (eid == iota_e).astype(jnp.float32)
        for c in range(NCLS):
            mc = (cls == c).astype(jnp.float32)
            hist[:, c:c + 1] += jnp.sum(oh * mc, axis=1, keepdims=True)


def _run_stage_b1(eid, w):
    return pl.pallas_call(
        _stage_b1_body,
        grid=(NB + 1,),
        in_specs=[
            pl.BlockSpec((1, 1, RB), lambda i: (jnp.minimum(i, NB - 1), 0, 0)),
            pl.BlockSpec((1, 1, RB), lambda i: (jnp.minimum(i, NB - 1), 0, 0)),
        ],
        out_specs=pl.BlockSpec((1, EE, NCLS), lambda i: (i, 0, 0)),
        out_shape=jax.ShapeDtypeStruct((NB + 1, EE, NCLS), jnp.float32),
        scratch_shapes=[pltpu.VMEM((EE, NCLS), jnp.float32)],
    )(eid, w)


def _stage_b2_body(eid_ref, w_ref, pfx_ref, tot_ref, gidx_ref):
    eid = eid_ref[...].reshape(1, RB)
    cls = _cls_of(w_ref[...].reshape(1, RB))
    iota_e = lax.broadcasted_iota(jnp.int32, (EE, 1), 0)
    oh = (eid == iota_e).astype(jnp.float32)
    ia = lax.broadcasted_iota(jnp.int32, (RB, RB), 0)
    ib = lax.broadcasted_iota(jnp.int32, (RB, RB), 1)
    utri = (ia < ib).astype(jnp.float32)

    pfx = pfx_ref[0]
    tot = tot_ref[0]
    base = jnp.zeros((1, RB), jnp.float32)
    within = jnp.zeros((1, RB), jnp.float32)
    higher = jnp.zeros((EE, 1), jnp.float32)
    for c in range(NCLS):
        row = higher + pfx[:, c:c + 1]
        mc = (cls == c).astype(jnp.float32)
        base += mc * jnp.sum(oh * row, axis=0, keepdims=True)
        ohc = oh * mc
        cumc = lax.dot_general(ohc, utri, (((1,), (0,)), ((), ())),
                               preferred_element_type=jnp.float32)
        within += jnp.sum(ohc * cumc, axis=0, keepdims=True)
        higher = higher + tot[:, c:c + 1]
    slot = base + within
    keep = slot < float(CAP)
    gidx = jnp.where(keep, eid.astype(jnp.float32) * CAP + slot,
                     float(EE * CAP)).astype(jnp.int32)
    gidx_ref[...] = gidx.reshape(1, 1, RB)


def _run_stage_b2(eid, w, pfx):
    return pl.pallas_call(
        _stage_b2_body,
        grid=(NB,),
        in_specs=[
            pl.BlockSpec((1, 1, RB), lambda i: (i, 0, 0)),
            pl.BlockSpec((1, 1, RB), lambda i: (i, 0, 0)),
            pl.BlockSpec((1, EE, NCLS), lambda i: (i, 0, 0)),
            pl.BlockSpec((1, EE, NCLS), lambda i: (NB, 0, 0)),
        ],
        out_specs=pl.BlockSpec((1, 1, RB), lambda i: (i, 0, 0)),
        out_shape=jax.ShapeDtypeStruct((NB, 1, RB), jnp.int32),
    )(eid, w, pfx, pfx)


def _sc_dispatch_body(comb_hbm, gidx_hbm, disp_hbm, idx_v, rows_v, sem):
    wid = lax.axis_index("s") * 2 + lax.axis_index("c")
    rows_per_tile = TT // 32

    def body(j, carry):
        base = pl.multiple_of(wid * rows_per_tile + j * CHUNK, CHUNK)
        pltpu.sync_copy(gidx_hbm.at[pl.ds(base, CHUNK)], idx_v)
        pltpu.sync_copy(comb_hbm.at[pl.ds(PREFIX + base, CHUNK)], rows_v)
        pltpu.async_copy(rows_v, disp_hbm.at[idx_v], sem).wait()
        return carry

    lax.fori_loop(0, rows_per_tile // CHUNK, body, 0)


def _run_stage_c(comb, gidx_flat):
    mesh = plsc.VectorSubcoreMesh(core_axis_name="c", subcore_axis_name="s")
    f = pl.kernel(
        _sc_dispatch_body,
        out_type=jax.ShapeDtypeStruct((NDISP, DD), jnp.float32),
        mesh=mesh,
        scratch_types=[
            pltpu.VMEM((CHUNK,), jnp.int32),
            pltpu.VMEM((CHUNK, DD), jnp.float32),
            pltpu.SemaphoreType.DMA,
        ],
    )
    return f(comb, gidx_flat)


def _stage_d_body(comb_in_ref, disp_ref, w1_ref, b1_ref, w2_ref, b2_ref,
                  comb_out_ref):
    del comb_in_ref
    xb = disp_ref[...]
    u = lax.dot_general(
        xb.astype(jnp.bfloat16), w1_ref[0].astype(jnp.bfloat16),
        (((1,), (0,)), ((), ())), preferred_element_type=jnp.float32)
    u = u + b1_ref[0]
    h = jax.nn.gelu(u)
    o = lax.dot_general(
        h.astype(jnp.bfloat16), w2_ref[0].astype(jnp.bfloat16),
        (((1,), (0,)), ((), ())), preferred_element_type=jnp.float32)
    comb_out_ref[...] = o + b2_ref[0]


def _run_stage_d(comb, disp, W1, b1, W2, b2):
    return pl.pallas_call(
        _stage_d_body,
        grid=(EE,),
        in_specs=[
            pl.BlockSpec(memory_space=pl.ANY),
            pl.BlockSpec((CAP, DD), lambda e: (e, 0)),
            pl.BlockSpec((1, DD, DFF), lambda e: (e, 0, 0)),
            pl.BlockSpec((1, 1, DFF), lambda e: (e, 0, 0)),
            pl.BlockSpec((1, DFF, OUT), lambda e: (e, 0, 0)),
            pl.BlockSpec((1, 1, OUT), lambda e: (e, 0, 0)),
        ],
        out_specs=pl.BlockSpec((CAP, OUT), lambda e: (e, 0)),
        out_shape=jax.ShapeDtypeStruct((NC, OUT), jnp.float32),
        input_output_aliases={0: 0},
    )(comb, disp, W1, b1.reshape(EE, 1, DFF), W2, b2.reshape(EE, 1, OUT))


def _sc_combine_body(comb_hbm, gidx_hbm, out_hbm, idx_v, g3_v, rows_v, sem):
    wid = lax.axis_index("s") * 2 + lax.axis_index("c")
    rows_per_tile = TT // 32

    def body(j, carry):
        base = pl.multiple_of(wid * rows_per_tile + j * CHUNK, CHUNK)
        pltpu.sync_copy(gidx_hbm.at[pl.ds(base, CHUNK)], idx_v)
        for i in range(CHUNK // 16):
            v = idx_v[pl.ds(i * 16, 16)]
            tok = base + i * 16 + lax.iota(jnp.int32, 16)
            g3_v[pl.ds(i * 16, 16)] = jnp.where(v < EE * CAP, v, PREFIX + tok)
        pltpu.async_copy(comb_hbm.at[g3_v], rows_v, sem).wait()
        pltpu.sync_copy(rows_v, out_hbm.at[pl.ds(base, CHUNK)])
        return carry

    lax.fori_loop(0, rows_per_tile // CHUNK, body, 0)


def _run_stage_e(comb2, gidx_flat):
    mesh = plsc.VectorSubcoreMesh(core_axis_name="c", subcore_axis_name="s")
    f = pl.kernel(
        _sc_combine_body,
        out_type=jax.ShapeDtypeStruct((TT, OUT), jnp.float32),
        mesh=mesh,
        scratch_types=[
            pltpu.VMEM((CHUNK,), jnp.int32),
            pltpu.VMEM((CHUNK,), jnp.int32),
            pltpu.VMEM((CHUNK, OUT), jnp.float32),
            pltpu.SemaphoreType.DMA,
        ],
    )
    return f(comb2, gidx_flat)


def kernel(x, gate_W, gate_b, ln_scale, ln_bias, W1, b1, W2, b2):
    xf = x.reshape(TT, DD)
    noise = NOISE_STD * jax.random.normal(
        jax.random.key(42), (TT, EE), dtype=jnp.float32)

    comb, aux = _run_stage_a(xf, noise, gate_W, gate_b, ln_scale, ln_bias)

    mu = xf.mean(-1, keepdims=True)
    var = ((xf - mu) ** 2).mean(-1, keepdims=True)
    xn_r = (xf - mu) / jnp.sqrt(var + 1e-5) * ln_scale + ln_bias
    logits = xn_r @ gate_W + gate_b
    gates_noisy = jax.nn.softmax(logits + noise, axis=-1)
    gates_noisy = lax.optimization_barrier(gates_noisy)
    topi = jnp.argmax(gates_noisy, axis=-1, keepdims=True)
    topv = jnp.take_along_axis(gates_noisy, topi, axis=-1)
    wsel = (topv / (topv.sum(-1, keepdims=True) + 1e-20)).reshape(NB, 1, RB)
    eid = topi.astype(jnp.int32).reshape(NB, 1, RB)

    pfx = _run_stage_b1(eid, wsel)
    gidx = _run_stage_b2(eid, wsel, pfx)
    gidx_flat = gidx.reshape(TT)
    disp = _run_stage_c(comb, gidx_flat)
    comb2 = _run_stage_d(comb, disp, W1, b1, W2, b2)
    out = _run_stage_e(comb2, gidx_flat)
    return out.reshape(BB, SS, OUT), aux[0, 0]


# speedup vs baseline: 6.7775x; 1.0142x over previous
import jax
import jax.numpy as jnp
from jax import lax
from jax.experimental import pallas as pl
from jax.experimental.pallas import tpu as pltpu
from jax.experimental.pallas import tpu_sc as plsc

BB, SS, DD = 4, 8192, 768
EE, DFF, OUT = 64, 64, 768
TT = BB * SS
CAP = TT // EE
NOISE_STD = 1.0 / EE
RA = 2048
NA = TT // RA
RB = 512
NB = TT // RB
NCLS = 8
PAD = 2048
PREFIX = EE * CAP + PAD
NC = PREFIX + TT
NDISP = EE * CAP + 8
CHUNK = 64
SQRT_HALF = 0.7071067811865476


def _stage_a_body(x_ref, noise_ref, gw_ref, gb_ref, ls_ref, lb_ref,
                  comb_ref, aux_ref, imp_acc, p_acc):
    i = pl.program_id(0)
    xb = x_ref[...]
    mu = jnp.mean(xb, axis=1, keepdims=True)
    xc = xb - mu
    var = jnp.mean(xc * xc, axis=1, keepdims=True)
    xn = xc / jnp.sqrt(var + 1e-5) * ls_ref[...] + lb_ref[...]
    comb_ref[...] = xn
    logits = lax.dot_general(
        xn.astype(jnp.bfloat16), gw_ref[...].astype(jnp.bfloat16),
        (((1,), (0,)), ((), ())), preferred_element_type=jnp.float32)
    logits = logits + gb_ref[...]

    @pl.when(i == 0)
    def _init():
        imp_acc[...] = jnp.zeros_like(imp_acc)
        p_acc[...] = jnp.zeros_like(p_acc)

    m = jnp.max(logits, axis=1, keepdims=True)
    ex = jnp.exp(logits - m)
    gates = ex / jnp.sum(ex, axis=1, keepdims=True)
    imp_acc[...] += jnp.sum(gates, axis=0, keepdims=True)

    noisy = logits + noise_ref[...]
    thr = jnp.max(noisy, axis=1, keepdims=True)
    z = (thr - logits) * (1.0 / NOISE_STD)
    p = 0.5 * (1.0 - lax.erf(z * SQRT_HALF))
    p_acc[...] += jnp.sum(p, axis=0, keepdims=True)

    @pl.when(i == NA - 1)
    def _fin():
        def cv2(v):
            mean = jnp.mean(v)
            ss = jnp.sum((v - mean) ** 2) / (EE - 1)
            return ss / (mean + 1e-8) ** 2
        imp_loss = cv2(imp_acc[...])
        load_loss = cv2(p_acc[...] / TT)
        aux_ref[...] = jnp.full((1, 1), 0.5, jnp.float32) * (imp_loss + load_loss)


def _run_stage_a(xf, noise, gate_W, gate_b, ln_scale, ln_bias):
    return pl.pallas_call(
        _stage_a_body,
        grid=(NA,),
        in_specs=[
            pl.BlockSpec((RA, DD), lambda i: (i, 0)),
            pl.BlockSpec((RA, EE), lambda i: (i, 0)),
            pl.BlockSpec((DD, EE), lambda i: (0, 0)),
            pl.BlockSpec((1, EE), lambda i: (0, 0)),
            pl.BlockSpec((1, DD), lambda i: (0, 0)),
            pl.BlockSpec((1, DD), lambda i: (0, 0)),
        ],
        out_specs=[
            pl.BlockSpec((RA, DD), lambda i: (i + PREFIX // RA, 0)),
            pl.BlockSpec((1, 1), lambda i: (0, 0)),
        ],
        out_shape=[
            jax.ShapeDtypeStruct((NC, DD), jnp.float32),
            jax.ShapeDtypeStruct((1, 1), jnp.float32),
        ],
        scratch_shapes=[
            pltpu.VMEM((1, EE), jnp.float32),
            pltpu.VMEM((1, EE), jnp.float32),
        ],
    )(xf, noise, gate_W, gate_b.reshape(1, EE),
      ln_scale.reshape(1, DD), ln_bias.reshape(1, DD))


def _cls_of(w):
    one = jnp.int32(0x3F800000)
    k = one - lax.bitcast_convert_type(w, jnp.int32)
    return jnp.clip(k + 1, 0, NCLS - 1)


def _stage_b1_body(eid_ref, w_ref, pfx_ref, hist):
    i = pl.program_id(0)

    @pl.when(i == 0)
    def _init():
        hist[...] = jnp.zeros_like(hist)

    pfx_ref[...] = hist[...].reshape(1, EE, NCLS)

    @pl.when(i < NB)
    def _accum():
        eid = eid_ref[...].reshape(1, RB)
        cls = _cls_of(w_ref[...].reshape(1, RB))
        iota_e = lax.broadcasted_iota(jnp.int32, (EE, 1), 0)
        oh = (eid == iota_e).astype(jnp.float32)
        for c in range(NCLS):
            mc = (cls == c).astype(jnp.float32)
            hist[:, c:c + 1] += jnp.sum(oh * mc, axis=1, keepdims=True)


def _run_stage_b1(eid, w):
    return pl.pallas_call(
        _stage_b1_body,
        grid=(NB + 1,),
        in_specs=[
            pl.BlockSpec((1, 1, RB), lambda i: (jnp.minimum(i, NB - 1), 0, 0)),
            pl.BlockSpec((1, 1, RB), lambda i: (jnp.minimum(i, NB - 1), 0, 0)),
        ],
        out_specs=pl.BlockSpec((1, EE, NCLS), lambda i: (i, 0, 0)),
        out_shape=jax.ShapeDtypeStruct((NB + 1, EE, NCLS), jnp.float32),
        scratch_shapes=[pltpu.VMEM((EE, NCLS), jnp.float32)],
    )(eid, w)


def _stage_b2_body(eid_ref, w_ref, pfx_ref, tot_ref, gidx_ref):
    eid = eid_ref[...].reshape(1, RB)
    cls = _cls_of(w_ref[...].reshape(1, RB))
    iota_e = lax.broadcasted_iota(jnp.int32, (EE, 1), 0)
    oh = (eid == iota_e).astype(jnp.float32)
    ia = lax.broadcasted_iota(jnp.int32, (RB, RB), 0)
    ib = lax.broadcasted_iota(jnp.int32, (RB, RB), 1)
    utri = (ia < ib).astype(jnp.float32)

    pfx = pfx_ref[0]
    tot = tot_ref[0]
    base = jnp.zeros((1, RB), jnp.float32)
    within = jnp.zeros((1, RB), jnp.float32)
    higher = jnp.zeros((EE, 1), jnp.float32)
    for c in range(NCLS):
        row = higher + pfx[:, c:c + 1]
        mc = (cls == c).astype(jnp.float32)
        base += mc * jnp.sum(oh * row, axis=0, keepdims=True)
        ohc = oh * mc
        cumc = lax.dot_general(ohc, utri, (((1,), (0,)), ((), ())),
                               preferred_element_type=jnp.float32)
        within += jnp.sum(ohc * cumc, axis=0, keepdims=True)
        higher = higher + tot[:, c:c + 1]
    slot = base + within
    keep = slot < float(CAP)
    gidx = jnp.where(keep, eid.astype(jnp.float32) * CAP + slot,
                     float(EE * CAP)).astype(jnp.int32)
    gidx_ref[...] = gidx.reshape(1, 1, RB)


def _run_stage_b2(eid, w, pfx):
    return pl.pallas_call(
        _stage_b2_body,
        grid=(NB,),
        in_specs=[
            pl.BlockSpec((1, 1, RB), lambda i: (i, 0, 0)),
            pl.BlockSpec((1, 1, RB), lambda i: (i, 0, 0)),
            pl.BlockSpec((1, EE, NCLS), lambda i: (i, 0, 0)),
            pl.BlockSpec((1, EE, NCLS), lambda i: (NB, 0, 0)),
        ],
        out_specs=pl.BlockSpec((1, 1, RB), lambda i: (i, 0, 0)),
        out_shape=jax.ShapeDtypeStruct((NB, 1, RB), jnp.int32),
    )(eid, w, pfx, pfx)


def _sc_dispatch_body(comb_hbm, gidx_hbm, disp_hbm,
                      idx0, idx1, rows0, rows1, lsem0, lsem1, ssem0, ssem1):
    wid = lax.axis_index("s") * 2 + lax.axis_index("c")
    rows_per_tile = TT // 32
    nch = rows_per_tile // CHUNK

    def start_load(j, idx_v, rows_v, lsem):
        base = pl.multiple_of(wid * rows_per_tile + j * CHUNK, CHUNK)
        pltpu.async_copy(gidx_hbm.at[pl.ds(base, CHUNK)], idx_v, lsem)
        pltpu.async_copy(comb_hbm.at[pl.ds(PREFIX + base, CHUNK)], rows_v, lsem)

    def wait_load(idx_v, rows_v, lsem):
        pltpu.make_async_copy(gidx_hbm.at[pl.ds(0, CHUNK)], idx_v, lsem).wait()
        pltpu.make_async_copy(comb_hbm.at[pl.ds(0, CHUNK)], rows_v, lsem).wait()

    start_load(0, idx0, rows0, lsem0)

    def body(j2, carry):
        wait_load(idx0, rows0, lsem0)
        start_load(2 * j2 + 1, idx1, rows1, lsem1)
        sc0 = pltpu.async_copy(rows0, disp_hbm.at[idx0], ssem0)
        wait_load(idx1, rows1, lsem1)
        sc0.wait()

        @pl.when(2 * j2 + 2 < nch)
        def _():
            start_load(2 * j2 + 2, idx0, rows0, lsem0)

        pltpu.async_copy(rows1, disp_hbm.at[idx1], ssem1).wait()
        return carry

    lax.fori_loop(0, nch // 2, body, 0)


def _run_stage_c(comb, gidx_flat):
    mesh = plsc.VectorSubcoreMesh(core_axis_name="c", subcore_axis_name="s")
    f = pl.kernel(
        _sc_dispatch_body,
        out_type=jax.ShapeDtypeStruct((NDISP, DD), jnp.float32),
        mesh=mesh,
        scratch_types=[
            pltpu.VMEM((CHUNK,), jnp.int32),
            pltpu.VMEM((CHUNK,), jnp.int32),
            pltpu.VMEM((CHUNK, DD), jnp.float32),
            pltpu.VMEM((CHUNK, DD), jnp.float32),
            pltpu.SemaphoreType.DMA,
            pltpu.SemaphoreType.DMA,
            pltpu.SemaphoreType.DMA,
            pltpu.SemaphoreType.DMA,
        ],
    )
    return f(comb, gidx_flat)


def _stage_d_body(comb_in_ref, disp_ref, w1_ref, b1_ref, w2_ref, b2_ref,
                  comb_out_ref):
    del comb_in_ref
    xb = disp_ref[...]
    u = lax.dot_general(
        xb.astype(jnp.bfloat16), w1_ref[0].astype(jnp.bfloat16),
        (((1,), (0,)), ((), ())), preferred_element_type=jnp.float32)
    u = u + b1_ref[0]
    h = jax.nn.gelu(u)
    o = lax.dot_general(
        h.astype(jnp.bfloat16), w2_ref[0].astype(jnp.bfloat16),
        (((1,), (0,)), ((), ())), preferred_element_type=jnp.float32)
    comb_out_ref[...] = o + b2_ref[0]


def _run_stage_d(comb, disp, W1, b1, W2, b2):
    return pl.pallas_call(
        _stage_d_body,
        grid=(EE,),
        in_specs=[
            pl.BlockSpec(memory_space=pl.ANY),
            pl.BlockSpec((CAP, DD), lambda e: (e, 0)),
            pl.BlockSpec((1, DD, DFF), lambda e: (e, 0, 0)),
            pl.BlockSpec((1, 1, DFF), lambda e: (e, 0, 0)),
            pl.BlockSpec((1, DFF, OUT), lambda e: (e, 0, 0)),
            pl.BlockSpec((1, 1, OUT), lambda e: (e, 0, 0)),
        ],
        out_specs=pl.BlockSpec((CAP, OUT), lambda e: (e, 0)),
        out_shape=jax.ShapeDtypeStruct((NC, OUT), jnp.float32),
        input_output_aliases={0: 0},
    )(comb, disp, W1, b1.reshape(EE, 1, DFF), W2, b2.reshape(EE, 1, OUT))


def _sc_combine_body(comb_hbm, gidx_hbm, out_hbm, idx_v, g3_v, rows_v, sem):
    wid = lax.axis_index("s") * 2 + lax.axis_index("c")
    rows_per_tile = TT // 32

    def body(j, carry):
        base = pl.multiple_of(wid * rows_per_tile + j * CHUNK, CHUNK)
        pltpu.sync_copy(gidx_hbm.at[pl.ds(base, CHUNK)], idx_v)
        for i in range(CHUNK // 16):
            v = idx_v[pl.ds(i * 16, 16)]
            tok = base + i * 16 + lax.iota(jnp.int32, 16)
            g3_v[pl.ds(i * 16, 16)] = jnp.where(v < EE * CAP, v, PREFIX + tok)
        pltpu.async_copy(comb_hbm.at[g3_v], rows_v, sem).wait()
        pltpu.sync_copy(rows_v, out_hbm.at[pl.ds(base, CHUNK)])
        return carry

    lax.fori_loop(0, rows_per_tile // CHUNK, body, 0)


def _run_stage_e(comb2, gidx_flat):
    mesh = plsc.VectorSubcoreMesh(core_axis_name="c", subcore_axis_name="s")
    f = pl.kernel(
        _sc_combine_body,
        out_type=jax.ShapeDtypeStruct((TT, OUT), jnp.float32),
        mesh=mesh,
        scratch_types=[
            pltpu.VMEM((CHUNK,), jnp.int32),
            pltpu.VMEM((CHUNK,), jnp.int32),
            pltpu.VMEM((CHUNK, OUT), jnp.float32),
            pltpu.SemaphoreType.DMA,
        ],
    )
    return f(comb2, gidx_flat)


def kernel(x, gate_W, gate_b, ln_scale, ln_bias, W1, b1, W2, b2):
    xf = x.reshape(TT, DD)
    noise = NOISE_STD * jax.random.normal(
        jax.random.key(42), (TT, EE), dtype=jnp.float32)

    comb, aux = _run_stage_a(xf, noise, gate_W, gate_b, ln_scale, ln_bias)

    mu = xf.mean(-1, keepdims=True)
    var = ((xf - mu) ** 2).mean(-1, keepdims=True)
    xn_r = (xf - mu) / jnp.sqrt(var + 1e-5) * ln_scale + ln_bias
    logits = xn_r @ gate_W + gate_b
    gates_noisy = jax.nn.softmax(logits + noise, axis=-1)
    gates_noisy = lax.optimization_barrier(gates_noisy)
    topi = jnp.argmax(gates_noisy, axis=-1, keepdims=True)
    topv = jnp.take_along_axis(gates_noisy, topi, axis=-1)
    wsel = (topv / (topv.sum(-1, keepdims=True) + 1e-20)).reshape(NB, 1, RB)
    eid = topi.astype(jnp.int32).reshape(NB, 1, RB)

    pfx = _run_stage_b1(eid, wsel)
    gidx = _run_stage_b2(eid, wsel, pfx)
    gidx_flat = gidx.reshape(TT)
    disp = _run_stage_c(comb, gidx_flat)
    comb2 = _run_stage_d(comb, disp, W1, b1, W2, b2)
    out = _run_stage_e(comb2, gidx_flat)
    return out.reshape(BB, SS, OUT), aux[0, 0]
